# baseline (speedup 1.0000x reference)
import jax
import jax.numpy as jnp
from jax import lax
from jax.experimental import pallas as pl
from jax.experimental.pallas import tpu as pltpu
from jax.experimental.pallas import tpu_sc as plsc

_R0, _C0, _R1, _C1 = 1024, 1024, 3071, 3071
_N = 4096
_NC, _NS = 2, 16
_NW = _NC * _NS
_RPW = _N // _NW
_W = _C1 - _C0 + 1
_FR = 8
_ECR = 4
_ICR = 16
_EROWS = 108
_XROWS = (_R0 - 8 * _EROWS) // 8


def _fw_ring(b_hbm, out_hbm, start, nchunks, vb, sin, sout):
    hin = [None] * 3
    hout = [None] * 3
    for k in range(3):
        hin[k] = pltpu.async_copy(
            b_hbm.at[pl.ds(start + k * _ECR, _ECR)], vb[k], sin[k])
    for k in range(nchunks):
        b = k % 3
        hin[b].wait()
        hout[b] = pltpu.async_copy(
            vb[b], out_hbm.at[pl.ds(start + k * _ECR, _ECR)], sout[b])
        if k + 3 < nchunks:
            hout[b].wait()
            hin[b] = pltpu.async_copy(
                b_hbm.at[pl.ds(start + (k + 3) * _ECR, _ECR)], vb[b], sin[b])
    hout[(nchunks - 3) % 3].wait()
    hout[(nchunks - 2) % 3].wait()
    hout[(nchunks - 1) % 3].wait()


def _sc_kernel(b_hbm, bt_hbm, out_hbm, btv, fillv, vb0, vb1, vb2, sb0, sb1,
               sb2, si0, si1, si2, so0, so1, so2, sfill):
    cid = lax.axis_index("c")
    sid = lax.axis_index("s")
    wid = sid * _NC + cid
    interior = jnp.logical_and(wid >= 8, wid < 24)
    vb = [vb0, vb1, vb2]
    sin = [si0, si1, si2]
    sout = [so0, so1, so2]

    @pl.when(jnp.logical_not(interior))
    def _exterior():
        est = jnp.where(wid < 8, wid * _EROWS,
                        _C1 + 1 + 8 * _XROWS + (wid - 24) * _EROWS)
        _fw_ring(b_hbm, out_hbm, est, _EROWS // _ECR, vb, sin, sout)

    @pl.when(interior)
    def _interior():
        i = wid - 8
        r0 = wid * _RPW
        cols = [0, _C1 + 1]
        chunks = [(r0 + (j // 2) * _ICR, cols[j & 1])
                  for j in range(2 * (_RPW // _ICR))]
        n = len(chunks)
        sb = [sb0, sb1, sb2]
        hin = [None] * 3
        hout = [None] * 3
        for k in range(3):
            rr, cc = chunks[k]
            hin[k] = pltpu.async_copy(
                b_hbm.at[pl.ds(rr, _ICR), pl.ds(cc, _C0)], sb[k], sin[k])
        pltpu.sync_copy(bt_hbm, btv)
        v = btv[...]
        for r in range(_FR):
            for q in range(_W // 16):
                fillv[r, pl.ds(q * 16, 16)] = v
        fills = [
            pltpu.async_copy(
                fillv, out_hbm.at[pl.ds(r0 + k * _FR, _FR), pl.ds(_C0, _W)],
                sfill)
            for k in range(_RPW // _FR)
        ]
        for k in range(n):
            b = k % 3
            rr, cc = chunks[k]
            hin[b].wait()
            hout[b] = pltpu.async_copy(
                sb[b], out_hbm.at[pl.ds(rr, _ICR), pl.ds(cc, _C0)], sout[b])
            if k + 3 < n:
                hout[b].wait()
                rr2, cc2 = chunks[k + 3]
                hin[b] = pltpu.async_copy(
                    b_hbm.at[pl.ds(rr2, _ICR), pl.ds(cc2, _C0)], sb[b], sin[b])
        hout[(n - 3) % 3].wait()
        hout[(n - 2) % 3].wait()
        hout[(n - 1) % 3].wait()
        xst = jnp.where(i < 8, 8 * _EROWS + i * _XROWS,
                        _C1 + 1 + (i - 8) * _XROWS)
        _fw_ring(b_hbm, out_hbm, xst, _XROWS // _ECR, vb, sin, sout)
        for c in fills:
            c.wait()


@jax.jit
def _run(b2, bt16):
    mesh = plsc.VectorSubcoreMesh(core_axis_name="c", subcore_axis_name="s",
                                  num_cores=_NC, num_subcores=_NS)
    return pl.kernel(
        _sc_kernel,
        out_type=jax.ShapeDtypeStruct((_N, _N), jnp.float32),
        mesh=mesh,
        scratch_types=[
            pltpu.VMEM((16,), jnp.float32),
            pltpu.VMEM((_FR, _W), jnp.float32),
            pltpu.VMEM((_ECR, _N), jnp.float32),
            pltpu.VMEM((_ECR, _N), jnp.float32),
            pltpu.VMEM((_ECR, _N), jnp.float32),
            pltpu.VMEM((_ICR, _C0), jnp.float32),
            pltpu.VMEM((_ICR, _C0), jnp.float32),
            pltpu.VMEM((_ICR, _C0), jnp.float32),
            pltpu.SemaphoreType.DMA,
            pltpu.SemaphoreType.DMA,
            pltpu.SemaphoreType.DMA,
            pltpu.SemaphoreType.DMA,
            pltpu.SemaphoreType.DMA,
            pltpu.SemaphoreType.DMA,
            pltpu.SemaphoreType.DMA,
        ],
    )(b2, bt16)


def kernel(B, Bt):
    b2 = B.reshape(_N, _N)
    bt16 = jnp.broadcast_to(Bt.reshape(1), (16,))
    return _run(b2, bt16).reshape(1, _N, _N)
